# Initial kernel scaffold; baseline (speedup 1.0000x reference)
import jax
import jax.numpy as jnp
from jax.experimental import pallas as pl


def kernel(full_board_vector, piece_ids, Wm, bv):
    raise NotImplementedError("write your pallas kernel here")


# speedup vs baseline: 11.9285x; 11.9285x over previous
import functools

import jax
import jax.numpy as jnp
from jax.experimental import pallas as pl


def _tc_body(ids_ref, board_ref, wt_ref, bv_ref, out_ref, *, bblk, P, C, F, HW):
    idsb = ids_ref[...]
    pos = jax.lax.broadcasted_iota(jnp.int32, (bblk, P, HW), 2)
    piece = jax.lax.broadcasted_iota(jnp.int32, (bblk, P, HW), 1) + 1
    eq = idsb[:, None, :] == piece
    masked_pos = jnp.where(eq, pos, HW)
    first = jnp.min(masked_pos, axis=2, keepdims=True)
    fmask = (pos == first).astype(jnp.float32)
    acc = jnp.broadcast_to(bv_ref[0:1, :][None], (bblk, P, F))
    for c in range(C):
        bc = board_ref[:, c, :]
        g = jnp.sum(fmask * bc[:, None, :], axis=2, keepdims=True)
        acc = acc + g * wt_ref[c : c + 1, :][None]
    out_ref[...] = acc


def kernel(full_board_vector, piece_ids, Wm, bv):
    B, C, H, W = full_board_vector.shape
    HW = H * W
    F = Wm.shape[0]
    P = 32
    ids = piece_ids.reshape(B, HW).astype(jnp.int32)
    board = full_board_vector.reshape(B, C, HW)
    wt = Wm.T
    bv2 = bv.reshape(1, F)

    BBLK = 256
    grid = (B // BBLK,)
    body = functools.partial(_tc_body, bblk=BBLK, P=P, C=C, F=F, HW=HW)
    out = pl.pallas_call(
        body,
        grid=grid,
        in_specs=[
            pl.BlockSpec((BBLK, HW), lambda i: (i, 0)),
            pl.BlockSpec((BBLK, C, HW), lambda i: (i, 0, 0)),
            pl.BlockSpec((C, F), lambda i: (0, 0)),
            pl.BlockSpec((1, F), lambda i: (0, 0)),
        ],
        out_specs=pl.BlockSpec((BBLK, P, F), lambda i: (i, 0, 0)),
        out_shape=jax.ShapeDtypeStruct((B, P, F), jnp.float32),
    )(ids, board, wt, bv2)
    return out


# speedup vs baseline: 52.9735x; 4.4409x over previous
import functools

import jax
import jax.numpy as jnp
from jax.experimental import pallas as pl


def _tc_body(idst_ref, board_ref, bigw_ref, bias_ref, out_ref, *, bblk, P, C, F, HW):
    bt = jnp.transpose(board_ref[...])
    piece = jax.lax.broadcasted_iota(jnp.int32, (P, bblk), 0) + 1
    halves = [range(0, C // 2), range(C // 2, C)]
    parts = []
    for chans in halves:
        G = {c: jnp.zeros((P, bblk), jnp.float32) for c in chans}
        for hw in range(HW - 1, -1, -1):
            idrow = idst_ref[hw : hw + 1, :]
            mask = idrow == piece
            for c in chans:
                brow = bt[c * HW + hw : c * HW + hw + 1, :]
                G[c] = jnp.where(mask, brow, G[c])
        parts.extend(G[c] for c in chans)
    Gcat = jnp.concatenate(parts, axis=0)
    out = jax.lax.dot_general(
        Gcat, bigw_ref[...], (((0,), (0,)), ((), ())),
        preferred_element_type=jnp.float32,
    )
    out_ref[...] = out + bias_ref[...]


def kernel(full_board_vector, piece_ids, Wm, bv):
    B, C, H, W = full_board_vector.shape
    HW = H * W
    F = Wm.shape[0]
    P = 32
    ids_t = piece_ids.reshape(B, HW).astype(jnp.int32).T
    board = full_board_vector.reshape(B, C * HW)
    eye = jnp.eye(P, dtype=jnp.float32)
    bigw = jnp.einsum("pq,fc->cpqf", eye, Wm).reshape(C * P, P * F)
    bias = jnp.tile(bv, P).reshape(1, P * F)

    BBLK = 128
    grid = (B // BBLK,)
    body = functools.partial(_tc_body, bblk=BBLK, P=P, C=C, F=F, HW=HW)
    out = pl.pallas_call(
        body,
        grid=grid,
        in_specs=[
            pl.BlockSpec((HW, BBLK), lambda i: (0, i)),
            pl.BlockSpec((BBLK, C * HW), lambda i: (i, 0)),
            pl.BlockSpec((C * P, P * F), lambda i: (0, 0)),
            pl.BlockSpec((1, P * F), lambda i: (0, 0)),
        ],
        out_specs=pl.BlockSpec((BBLK, P * F), lambda i: (i, 0)),
        out_shape=jax.ShapeDtypeStruct((B, P * F), jnp.float32),
    )(ids_t, board, bigw, bias)
    return out.reshape(B, P, F)
